# baseline (speedup 1.0000x reference)
import functools

import jax
import jax.numpy as jnp
from jax import lax
from jax.experimental import pallas as pl
from jax.experimental.pallas import tpu as pltpu, tpu_sc as plsc

C = 128
K = 16
LANES = 16
CH = C // LANES
W = 2 * C


@functools.cache
def _make_tc_stats(n_pad: int):
    inv_c = 1.0 / C

    def body(x_ref, o_ref):
        xb = x_ref[...]
        m = jnp.sum(xb, axis=0) * inv_c
        v = jnp.sum(xb * xb, axis=0) * inv_c - m * m
        o_ref[0, :] = m
        o_ref[1, :] = v

    return pl.pallas_call(
        body, out_shape=jax.ShapeDtypeStruct((8, n_pad), jnp.float32)
    )


@functools.cache
def _make_sc_kernel(n_pad: int, npt: int):
    info = plsc.get_sparse_core_info()
    nc = info.num_cores
    mesh = plsc.VectorSubcoreMesh(core_axis_name="c", subcore_axis_name="s")
    inv_c = 1.0 / C
    c1 = 1e-6
    c2 = 1e-6

    take_dnums = lax.GatherDimensionNumbers(
        offset_dims=(), collapsed_slice_dims=(0,), start_index_map=(0,)
    )

    def lane_take(v, perm):
        return lax.gather(
            v, perm[:, None], take_dnums, slice_sizes=(1,),
            mode=lax.GatherScatterMode.PROMISE_IN_BOUNDS,
        )

    def allsum(v):
        lane = lax.iota(jnp.int32, LANES)
        for sh in (8, 4, 2, 1):
            v = v + lane_take(v, lax.bitwise_xor(lane, sh))
        return v

    DEPTH = 4

    @functools.partial(
        pl.kernel,
        out_type=jax.ShapeDtypeStruct((n_pad, C), jnp.float32),
        mesh=mesh,
        scratch_types=[
            pltpu.VMEM((npt * 2 * K,), jnp.int32),
            pltpu.VMEM((npt, C), jnp.float32),
            pltpu.VMEM((n_pad,), jnp.float32),
            pltpu.VMEM((n_pad,), jnp.float32),
        ]
        + [pltpu.VMEM((2 * K, W // 2), jnp.int32) for _ in range(DEPTH)]
        + [pltpu.SemaphoreType.DMA for _ in range(DEPTH)],
        compiler_params=pltpu.CompilerParams(needs_layout_passes=False),
    )
    def sc_kernel(tbl_hbm, eidx_hbm, m_hbm, v_hbm, out_hbm, eidx_v, out_v, m_v, v_v, *ring):
        rows = ring[:DEPTH]
        sems = ring[DEPTH:]
        wid = lax.axis_index("s") * nc + lax.axis_index("c")
        base = wid * npt
        pltpu.sync_copy(eidx_hbm.at[pl.ds(base * 2 * K, npt * 2 * K)], eidx_v)
        pltpu.sync_copy(m_hbm, m_v)
        pltpu.sync_copy(v_hbm, v_v)

        def issue(nn, b):
            pltpu.async_copy(tbl_hbm.at[eidx_v.at[pl.ds(nn * 2 * K, 2 * K)]], rows[b], sems[b])

        def slot_wait(b):
            pltpu.make_async_copy(tbl_hbm.at[pl.ds(0, 2 * K)], rows[b], sems[b]).wait()

        def compute(nn, rows_v):
            idx_i = eidx_v[pl.ds(nn * 2 * K, LANES)]
            idx_j = eidx_v[pl.ds(nn * 2 * K + K, LANES)]
            mi = plsc.load_gather(m_v, [idx_i])
            vi = plsc.load_gather(v_v, [idx_i])
            mj = plsc.load_gather(m_v, [idx_j])
            vj = plsc.load_gather(v_v, [idx_j])
            mimj = mi * mj
            num1 = 2.0 * mimj + c1
            den1 = mi * mi + mj * mj + c1
            den2 = vi + vj + c2
            r12 = num1 / (den1 * den2)

            def edge_body(kk, accs):
                bidx = jnp.full((LANES,), kk, jnp.int32)
                dotv = None
                dv = []
                out = []
                for cc in range(CH // 2):
                    ai = plsc.unpack(plsc.bitcast(rows_v[kk, pl.ds(cc * 16, 16)], jnp.bfloat16),
                                     format=plsc.PackFormat.INTERLEAVED)
                    aj = plsc.unpack(plsc.bitcast(rows_v[kk + K, pl.ds(cc * 16, 16)], jnp.bfloat16),
                                     format=plsc.PackFormat.INTERLEAVED)
                    d0 = ai[0] * aj[0] + ai[1] * aj[1]
                    dotv = d0 if dotv is None else dotv + d0
                    pi = plsc.unpack(plsc.bitcast(rows_v[kk, pl.ds(C // 2 + cc * 16, 16)], jnp.bfloat16),
                                     format=plsc.PackFormat.INTERLEAVED)
                    pj = plsc.unpack(plsc.bitcast(rows_v[kk + K, pl.ds(C // 2 + cc * 16, 16)], jnp.bfloat16),
                                     format=plsc.PackFormat.INTERLEAVED)
                    for h in range(2):
                        out.append(accs[2 * cc + h] + (pi[h] + pj[h]))
                        dv.append(jnp.abs(pi[h] - pj[h]))
                mimj_b = lane_take(mimj, bidx)
                r12_b = lane_take(r12, bidx)
                cov2 = 2.0 * (allsum(dotv) * inv_c - mimj_b) + c2
                sff = 1.0 - r12_b * cov2
                return tuple(out[cc] + dv[cc] * sff for cc in range(CH))

            zeros = tuple(jnp.zeros((LANES,), jnp.float32) for _ in range(CH))
            accs = lax.fori_loop(0, K, edge_body, zeros)
            for cc in range(CH):
                out_v[nn, pl.ds(cc * LANES, LANES)] = accs[cc]

        for b in range(DEPTH):
            issue(b, b)

        def outer(g, carry):
            for b in range(DEPTH):
                nn = g * DEPTH + b
                slot_wait(b)
                compute(nn, rows[b])

                @pl.when(nn + DEPTH < npt)
                def _():
                    issue(nn + DEPTH, b)

            return carry

        lax.fori_loop(0, npt // DEPTH, outer, 0)
        pltpu.sync_copy(out_v, out_hbm.at[pl.ds(base, npt)])

    return sc_kernel


def kernel(x, x_p, edge_index):
    n = x.shape[2]
    x2 = x[0, :, :, 0]
    xp2 = x_p[0, :, :, 0]
    e = edge_index[:, 0].astype(jnp.int32)
    eidx = jnp.concatenate([e[1], e[0]], axis=1)

    info = plsc.get_sparse_core_info()
    nw = info.num_cores * info.num_subcores
    npt = -(-n // (nw * 8)) * 8
    n_pad = npt * nw
    eidx = jnp.pad(eidx, ((0, n_pad - n), (0, 0))).reshape(-1)

    x2p = jnp.pad(x2, ((0, 0), (0, n_pad - n)))
    mv = _make_tc_stats(n_pad)(x2p)
    tbl = jnp.concatenate([x2.T, xp2.T], axis=1).astype(jnp.bfloat16)
    tbl = jax.lax.bitcast_convert_type(tbl.reshape(n, W // 2, 2), jnp.int32)
    out_t = _make_sc_kernel(n_pad, npt)(tbl, eidx, mv[0], mv[1])
    import numpy as np
    pos = np.empty((C,), np.int32)
    for cc in range(C // 32):
        for r in range(32):
            pos[cc * 32 + r] = cc * 32 + (r // 2 if r % 2 == 0 else 16 + r // 2)
    out_t = jnp.take(out_t, jnp.asarray(pos), axis=1)
    return out_t[:n].T[None, :, :, None]


# speedup vs baseline: 1.0506x; 1.0506x over previous
import functools

import jax
import jax.numpy as jnp
from jax import lax
from jax.experimental import pallas as pl
from jax.experimental.pallas import tpu as pltpu, tpu_sc as plsc

C = 128
K = 16
LANES = 16
CH = C // LANES
W = 2 * C


@functools.cache
def _make_tc_stats(n_pad: int):
    inv_c = 1.0 / C

    def body(x_ref, o_ref):
        xb = x_ref[...]
        m = jnp.sum(xb, axis=0) * inv_c
        v = jnp.sum(xb * xb, axis=0) * inv_c - m * m
        o_ref[0, :] = m
        o_ref[1, :] = v

    return pl.pallas_call(
        body, out_shape=jax.ShapeDtypeStruct((8, n_pad), jnp.float32)
    )


@functools.cache
def _make_sc_kernel(n_pad: int, npt_f: int, npt_s: int, fast_core: int):
    npt = npt_f
    info = plsc.get_sparse_core_info()
    nc = info.num_cores
    mesh = plsc.VectorSubcoreMesh(core_axis_name="c", subcore_axis_name="s")
    inv_c = 1.0 / C
    c1 = 1e-6
    c2 = 1e-6

    take_dnums = lax.GatherDimensionNumbers(
        offset_dims=(), collapsed_slice_dims=(0,), start_index_map=(0,)
    )

    def lane_take(v, perm):
        return lax.gather(
            v, perm[:, None], take_dnums, slice_sizes=(1,),
            mode=lax.GatherScatterMode.PROMISE_IN_BOUNDS,
        )

    def allsum(v):
        lane = lax.iota(jnp.int32, LANES)
        for sh in (8, 4, 2, 1):
            v = v + lane_take(v, lax.bitwise_xor(lane, sh))
        return v

    DEPTH = 8

    @functools.partial(
        pl.kernel,
        out_type=jax.ShapeDtypeStruct((n_pad, C), jnp.float32),
        mesh=mesh,
        scratch_types=[
            pltpu.VMEM((npt * 2 * K,), jnp.int32),
            pltpu.VMEM((npt, C), jnp.float32),
            pltpu.VMEM((n_pad,), jnp.float32),
            pltpu.VMEM((n_pad,), jnp.float32),
        ]
        + [pltpu.VMEM((2 * K, W // 2), jnp.int32) for _ in range(DEPTH)]
        + [pltpu.SemaphoreType.DMA for _ in range(DEPTH)],
        compiler_params=pltpu.CompilerParams(needs_layout_passes=False),
    )
    def sc_kernel(tbl_hbm, eidx_hbm, m_hbm, v_hbm, out_hbm, eidx_v, out_v, m_v, v_v, *ring):
        rows = ring[:DEPTH]
        sems = ring[DEPTH:]
        s_ax = lax.axis_index("s")
        c_ax = lax.axis_index("c")
        pltpu.sync_copy(m_hbm, m_v)
        pltpu.sync_copy(v_hbm, v_v)

        def issue(nn, b):
            pltpu.async_copy(tbl_hbm.at[eidx_v.at[pl.ds(nn * 2 * K, 2 * K)]], rows[b], sems[b])

        def slot_wait(b):
            pltpu.make_async_copy(tbl_hbm.at[pl.ds(0, 2 * K)], rows[b], sems[b]).wait()

        def compute(nn, rows_v):
            idx_i = eidx_v[pl.ds(nn * 2 * K, LANES)]
            idx_j = eidx_v[pl.ds(nn * 2 * K + K, LANES)]
            mi = plsc.load_gather(m_v, [idx_i])
            vi = plsc.load_gather(v_v, [idx_i])
            mj = plsc.load_gather(m_v, [idx_j])
            vj = plsc.load_gather(v_v, [idx_j])
            mimj = mi * mj
            num1 = 2.0 * mimj + c1
            den1 = mi * mi + mj * mj + c1
            den2 = vi + vj + c2
            r12 = num1 / (den1 * den2)

            def edge_body(kk, accs):
                bidx = jnp.full((LANES,), kk, jnp.int32)
                dotv = None
                dv = []
                out = []
                for cc in range(CH // 2):
                    ai = plsc.unpack(plsc.bitcast(rows_v[kk, pl.ds(cc * 16, 16)], jnp.bfloat16),
                                     format=plsc.PackFormat.INTERLEAVED)
                    aj = plsc.unpack(plsc.bitcast(rows_v[kk + K, pl.ds(cc * 16, 16)], jnp.bfloat16),
                                     format=plsc.PackFormat.INTERLEAVED)
                    d0 = ai[0] * aj[0] + ai[1] * aj[1]
                    dotv = d0 if dotv is None else dotv + d0
                    pi = plsc.unpack(plsc.bitcast(rows_v[kk, pl.ds(C // 2 + cc * 16, 16)], jnp.bfloat16),
                                     format=plsc.PackFormat.INTERLEAVED)
                    pj = plsc.unpack(plsc.bitcast(rows_v[kk + K, pl.ds(C // 2 + cc * 16, 16)], jnp.bfloat16),
                                     format=plsc.PackFormat.INTERLEAVED)
                    for h in range(2):
                        out.append(accs[2 * cc + h] + (pi[h] + pj[h]))
                        dv.append(jnp.abs(pi[h] - pj[h]))
                mimj_b = lane_take(mimj, bidx)
                r12_b = lane_take(r12, bidx)
                cov2 = 2.0 * (allsum(dotv) * inv_c - mimj_b) + c2
                sff = 1.0 - r12_b * cov2
                return tuple(out[cc] + dv[cc] * sff for cc in range(CH))

            zeros = tuple(jnp.zeros((LANES,), jnp.float32) for _ in range(CH))
            accs = lax.fori_loop(0, K, edge_body, zeros)
            for cc in range(CH):
                out_v[nn, pl.ds(cc * LANES, LANES)] = accs[cc]

        def run(base, npt_w):
            pltpu.sync_copy(
                eidx_hbm.at[pl.ds(base * 2 * K, npt_w * 2 * K)],
                eidx_v.at[pl.ds(0, npt_w * 2 * K)],
            )
            for b in range(DEPTH):
                issue(b, b)

            def outer(g, carry):
                for b in range(DEPTH):
                    nn = g * DEPTH + b
                    slot_wait(b)
                    compute(nn, rows[b])

                    @pl.when(nn + DEPTH < npt_w)
                    def _():
                        issue(nn + DEPTH, b)

                return carry

            lax.fori_loop(0, npt_w // DEPTH, outer, 0)
            pltpu.sync_copy(
                out_v.at[pl.ds(0, npt_w)], out_hbm.at[pl.ds(base, npt_w)]
            )

        nsub = 16

        @pl.when(c_ax == fast_core)
        def _():
            run(s_ax * npt_f, npt_f)

        @pl.when(c_ax != fast_core)
        def _():
            run(nsub * npt_f + s_ax * npt_s, npt_s)

    return sc_kernel


def kernel(x, x_p, edge_index):
    n = x.shape[2]
    x2 = x[0, :, :, 0]
    xp2 = x_p[0, :, :, 0]
    e = edge_index[:, 0].astype(jnp.int32)
    eidx = jnp.concatenate([e[1], e[0]], axis=1)

    info = plsc.get_sparse_core_info()
    nw = info.num_cores * info.num_subcores
    npt = -(-n // (nw * 8)) * 8
    n_pad = npt * nw
    eidx = jnp.pad(eidx, ((0, n_pad - n), (0, 0))).reshape(-1)

    npt_f = 480
    npt_s = 2 * npt - npt_f
    x2p = jnp.pad(x2, ((0, 0), (0, n_pad - n)))
    mv = _make_tc_stats(n_pad)(x2p)
    tbl = jnp.concatenate([x2.T, xp2.T], axis=1).astype(jnp.bfloat16)
    tbl = jax.lax.bitcast_convert_type(tbl.reshape(n, W // 2, 2), jnp.int32)
    out_t = _make_sc_kernel(n_pad, npt_f, npt_s, 0)(tbl, eidx, mv[0], mv[1])
    import numpy as np
    pos = np.empty((C,), np.int32)
    for cc in range(C // 32):
        for r in range(32):
            pos[cc * 32 + r] = cc * 32 + (r // 2 if r % 2 == 0 else 16 + r // 2)
    out_t = jnp.take(out_t, jnp.asarray(pos), axis=1)
    return out_t[:n].T[None, :, :, None]


# speedup vs baseline: 1.0835x; 1.0313x over previous
import functools

import jax
import jax.numpy as jnp
from jax import lax
from jax.experimental import pallas as pl
from jax.experimental.pallas import tpu as pltpu, tpu_sc as plsc

C = 128
K = 16
LANES = 16
CH = C // LANES
W = 2 * C


@functools.cache
def _make_tc_stats(n_pad: int):
    inv_c = 1.0 / C

    def body(x_ref, o_ref):
        xb = x_ref[...]
        m = jnp.sum(xb, axis=0) * inv_c
        v = jnp.sum(xb * xb, axis=0) * inv_c - m * m
        o_ref[0, :] = m
        o_ref[1, :] = v

    return pl.pallas_call(
        body, out_shape=jax.ShapeDtypeStruct((8, n_pad), jnp.float32)
    )


@functools.cache
def _make_sc_kernel(n_pad: int, npt_f: int, npt_s: int, fast_core: int):
    npt = npt_f
    info = plsc.get_sparse_core_info()
    nc = info.num_cores
    mesh = plsc.VectorSubcoreMesh(core_axis_name="c", subcore_axis_name="s")
    inv_c = 1.0 / C
    c1 = 1e-6
    c2 = 1e-6

    take_dnums = lax.GatherDimensionNumbers(
        offset_dims=(), collapsed_slice_dims=(0,), start_index_map=(0,)
    )

    def lane_take(v, perm):
        return lax.gather(
            v, perm[:, None], take_dnums, slice_sizes=(1,),
            mode=lax.GatherScatterMode.PROMISE_IN_BOUNDS,
        )

    def allsum(v):
        lane = lax.iota(jnp.int32, LANES)
        for sh in (8, 4, 2, 1):
            v = v + lane_take(v, lax.bitwise_xor(lane, sh))
        return v

    DEPTH = 8

    @functools.partial(
        pl.kernel,
        out_type=jax.ShapeDtypeStruct((n_pad, C), jnp.float32),
        mesh=mesh,
        scratch_types=[
            pltpu.VMEM((npt * 2 * K,), jnp.int32),
            pltpu.VMEM((npt, C), jnp.float32),
            pltpu.VMEM((n_pad,), jnp.float32),
            pltpu.VMEM((n_pad,), jnp.float32),
        ]
        + [pltpu.VMEM((2 * K, W // 2), jnp.int32) for _ in range(DEPTH)]
        + [pltpu.SemaphoreType.DMA for _ in range(DEPTH)],
        compiler_params=pltpu.CompilerParams(needs_layout_passes=False),
    )
    def sc_kernel(tbl_hbm, eidx_hbm, m_hbm, v_hbm, out_hbm, eidx_v, out_v, m_v, v_v, *ring):
        rows = ring[:DEPTH]
        sems = ring[DEPTH:]
        s_ax = lax.axis_index("s")
        c_ax = lax.axis_index("c")
        pltpu.sync_copy(m_hbm, m_v)
        pltpu.sync_copy(v_hbm, v_v)

        def issue(nn, b):
            pltpu.async_copy(tbl_hbm.at[eidx_v.at[pl.ds(nn * 2 * K, 2 * K)]], rows[b], sems[b])

        def slot_wait(b):
            pltpu.make_async_copy(tbl_hbm.at[pl.ds(0, 2 * K)], rows[b], sems[b]).wait()

        def compute(nn, rows_v):
            idx_i = eidx_v[pl.ds(nn * 2 * K, LANES)]
            idx_j = eidx_v[pl.ds(nn * 2 * K + K, LANES)]
            mi = plsc.load_gather(m_v, [idx_i])
            vi = plsc.load_gather(v_v, [idx_i])
            mj = plsc.load_gather(m_v, [idx_j])
            vj = plsc.load_gather(v_v, [idx_j])
            mimj = mi * mj
            num1 = 2.0 * mimj + c1
            den1 = mi * mi + mj * mj + c1
            den2 = vi + vj + c2
            r12 = num1 / (den1 * den2)

            def edge_body(kk, accs):
                bidx = jnp.full((LANES,), kk, jnp.int32)
                dotv = None
                dv = []
                out = []
                for cc in range(CH // 2):
                    ai = plsc.unpack(plsc.bitcast(rows_v[kk, pl.ds(cc * 16, 16)], jnp.bfloat16),
                                     format=plsc.PackFormat.INTERLEAVED)
                    aj = plsc.unpack(plsc.bitcast(rows_v[kk + K, pl.ds(cc * 16, 16)], jnp.bfloat16),
                                     format=plsc.PackFormat.INTERLEAVED)
                    d0 = ai[0] * aj[0] + ai[1] * aj[1]
                    dotv = d0 if dotv is None else dotv + d0
                    pi = plsc.unpack(plsc.bitcast(rows_v[kk, pl.ds(C // 2 + cc * 16, 16)], jnp.bfloat16),
                                     format=plsc.PackFormat.INTERLEAVED)
                    pj = plsc.unpack(plsc.bitcast(rows_v[kk + K, pl.ds(C // 2 + cc * 16, 16)], jnp.bfloat16),
                                     format=plsc.PackFormat.INTERLEAVED)
                    for h in range(2):
                        out.append(accs[2 * cc + h] + (pi[h] + pj[h]))
                        dv.append(jnp.abs(pi[h] - pj[h]))
                mimj_b = lane_take(mimj, bidx)
                r12_b = lane_take(r12, bidx)
                cov2 = 2.0 * (allsum(dotv) * inv_c - mimj_b) + c2
                sff = 1.0 - r12_b * cov2
                return tuple(out[cc] + dv[cc] * sff for cc in range(CH))

            zeros = tuple(jnp.zeros((LANES,), jnp.float32) for _ in range(CH))
            accs = lax.fori_loop(0, K, edge_body, zeros)
            for cc in range(CH):
                out_v[nn, pl.ds(cc * LANES, LANES)] = accs[cc]

        def run(base, npt_w):
            pltpu.sync_copy(
                eidx_hbm.at[pl.ds(base * 2 * K, npt_w * 2 * K)],
                eidx_v.at[pl.ds(0, npt_w * 2 * K)],
            )
            for b in range(DEPTH):
                issue(b, b)

            def outer(g, carry):
                for b in range(DEPTH):
                    nn = g * DEPTH + b
                    slot_wait(b)
                    compute(nn, rows[b])

                    @pl.when(nn + DEPTH < npt_w)
                    def _():
                        issue(nn + DEPTH, b)

                return carry

            lax.fori_loop(0, npt_w // DEPTH, outer, 0)
            pltpu.sync_copy(
                out_v.at[pl.ds(0, npt_w)], out_hbm.at[pl.ds(base, npt_w)]
            )

        nsub = 16
        half = nsub // 2
        on_fast = (s_ax // half) == fast_core
        fw = (s_ax % half) * nc + c_ax

        @pl.when(on_fast)
        def _():
            run(fw * npt_f, npt_f)

        @pl.when(jnp.logical_not(on_fast))
        def _():
            run(nsub * npt_f + fw * npt_s, npt_s)

    return sc_kernel


def kernel(x, x_p, edge_index):
    n = x.shape[2]
    x2 = x[0, :, :, 0]
    xp2 = x_p[0, :, :, 0]
    e = edge_index[:, 0].astype(jnp.int32)
    eidx = jnp.concatenate([e[1], e[0]], axis=1)

    info = plsc.get_sparse_core_info()
    nw = info.num_cores * info.num_subcores
    npt = -(-n // (nw * 8)) * 8
    n_pad = npt * nw
    eidx = jnp.pad(eidx, ((0, n_pad - n), (0, 0))).reshape(-1)

    npt_f = 480
    npt_s = 2 * npt - npt_f
    x2p = jnp.pad(x2, ((0, 0), (0, n_pad - n)))
    mv = _make_tc_stats(n_pad)(x2p)
    tbl = jnp.concatenate([x2.T, xp2.T], axis=1).astype(jnp.bfloat16)
    tbl = jax.lax.bitcast_convert_type(tbl.reshape(n, W // 2, 2), jnp.int32)
    out_t = _make_sc_kernel(n_pad, npt_f, npt_s, 0)(tbl, eidx, mv[0], mv[1])
    import numpy as np
    pos = np.empty((C,), np.int32)
    for cc in range(C // 32):
        for r in range(32):
            pos[cc * 32 + r] = cc * 32 + (r // 2 if r % 2 == 0 else 16 + r // 2)
    out_t = jnp.take(out_t, jnp.asarray(pos), axis=1)
    return out_t[:n].T[None, :, :, None]
